# Initial kernel scaffold; baseline (speedup 1.0000x reference)
import jax
import jax.numpy as jnp
from jax.experimental import pallas as pl


def kernel(nodes, edges, senders, receivers, spin_sites, node_graph_idx, W_en, b_en, W_ee, b_ee, W_msg, b_msg, W_node, b_node, W_edge, b_edge, W_p1, b_p1, W_p2, b_p2, W_v1, b_v1, W_v2, b_v2):
    raise NotImplementedError("write your pallas kernel here")


# speedup vs baseline: 1.7443x; 1.7443x over previous
import functools

import jax
import jax.numpy as jnp
from jax import lax
from jax.experimental import pallas as pl
from jax.experimental.pallas import tpu as pltpu
from jax.experimental.pallas import tpu_sc as plsc

N = 10000
E = 320000
DF = 128
DE = 16
DH = 128
NG = 100
NSS = 5
NC_OUT = 2
NL = 3

NCORE = 2
NSUB = 16
NW = NCORE * NSUB

_F32 = jnp.float32


def _dot(a, b):
    return lax.dot_general(a, b, (((1,), (0,)), ((), ())),
                           precision=lax.Precision.HIGHEST,
                           preferred_element_type=_F32)


def _ln(x):
    mu = jnp.mean(x, axis=-1, keepdims=True)
    var = jnp.mean((x - mu) ** 2, axis=-1, keepdims=True)
    return (x - mu) / jnp.sqrt(var + 1e-6)


def _enc_body(x_ref, w_ref, b_ref, o_ref):
    t = jax.nn.relu(_dot(x_ref[...], w_ref[...]) + b_ref[...])
    o_ref[...] = _ln(t)


@functools.lru_cache(maxsize=None)
def _make_encoder(rows, blk, din, dout):
    grid = rows // blk
    return pl.pallas_call(
        _enc_body,
        grid=(grid,),
        in_specs=[
            pl.BlockSpec((blk, din), lambda i: (i, 0)),
            pl.BlockSpec((din, dout), lambda i: (0, 0)),
            pl.BlockSpec((1, dout), lambda i: (0, 0)),
        ],
        out_specs=pl.BlockSpec((blk, dout), lambda i: (i, 0)),
        out_shape=jax.ShapeDtypeStruct((rows, dout), _F32),
    )


def _proj_body(h_ref, w_ref, ps_ref, pr_ref, hw_ref):
    t = _dot(h_ref[...], w_ref[...])
    ps_ref[...] = t[:, : 2 * DH]
    pr_ref[...] = t[:, 2 * DH: 4 * DH]
    hw_ref[...] = t[:, 4 * DH:]


@functools.lru_cache(maxsize=None)
def _make_proj(rows, blk):
    grid = rows // blk
    return pl.pallas_call(
        _proj_body,
        grid=(grid,),
        in_specs=[
            pl.BlockSpec((blk, DH), lambda i: (i, 0)),
            pl.BlockSpec((DH, 5 * DH), lambda i: (0, 0)),
        ],
        out_specs=[
            pl.BlockSpec((blk, 2 * DH), lambda i: (i, 0)),
            pl.BlockSpec((blk, 2 * DH), lambda i: (i, 0)),
            pl.BlockSpec((blk, DH), lambda i: (i, 0)),
        ],
        out_shape=[
            jax.ShapeDtypeStruct((rows, 2 * DH), _F32),
            jax.ShapeDtypeStruct((rows, 2 * DH), _F32),
            jax.ShapeDtypeStruct((rows, DH), _F32),
        ],
    )


def _combine_body(gs_ref, gr_ref, e_ref, w_ref, b_ref, m_ref, en_ref):
    e = e_ref[...]
    s = gs_ref[...] + gr_ref[...] + _dot(e, w_ref[...]) + b_ref[...]
    m_ref[...] = jax.nn.relu(s[:, :DH])
    en_ref[...] = _ln(jax.nn.relu(s[:, DH:]) + e)


@functools.lru_cache(maxsize=None)
def _make_combine(rows, blk):
    grid = rows // blk
    return pl.pallas_call(
        _combine_body,
        grid=(grid,),
        in_specs=[
            pl.BlockSpec((blk, 2 * DH), lambda i: (i, 0)),
            pl.BlockSpec((blk, 2 * DH), lambda i: (i, 0)),
            pl.BlockSpec((blk, DH), lambda i: (i, 0)),
            pl.BlockSpec((DH, 2 * DH), lambda i: (0, 0)),
            pl.BlockSpec((1, 2 * DH), lambda i: (0, 0)),
        ],
        out_specs=[
            pl.BlockSpec((blk, DH), lambda i: (i, 0)),
            pl.BlockSpec((blk, DH), lambda i: (i, 0)),
        ],
        out_shape=[
            jax.ShapeDtypeStruct((rows, DH), _F32),
            jax.ShapeDtypeStruct((rows, DH), _F32),
        ],
    )


def _node_upd_body(hw_ref, h_ref, agg_ref, wa_ref, b_ref, o_ref):
    h = h_ref[...]
    t = hw_ref[...] + _dot(agg_ref[...], wa_ref[...]) + b_ref[...]
    o_ref[...] = _ln(jax.nn.relu(t) + h)


@functools.lru_cache(maxsize=None)
def _make_node_update(rows, blk):
    grid = rows // blk
    return pl.pallas_call(
        _node_upd_body,
        grid=(grid,),
        in_specs=[
            pl.BlockSpec((blk, DH), lambda i: (i, 0)),
            pl.BlockSpec((blk, DH), lambda i: (i, 0)),
            pl.BlockSpec((blk, DH), lambda i: (i, 0)),
            pl.BlockSpec((DH, DH), lambda i: (0, 0)),
            pl.BlockSpec((1, DH), lambda i: (0, 0)),
        ],
        out_specs=pl.BlockSpec((blk, DH), lambda i: (i, 0)),
        out_shape=jax.ShapeDtypeStruct((rows, DH), _F32),
    )


def _segsum_body(h_ref, nig_ref, o_ref):
    i = pl.program_id(0)
    idx = nig_ref[0, 0, :]
    seg = lax.broadcasted_iota(jnp.int32, (NG, idx.shape[0]), 0)
    onehot = (seg == idx[None, :]).astype(_F32)
    acc = _dot(onehot, h_ref[...])

    @pl.when(i == 0)
    def _():
        o_ref[...] = acc

    @pl.when(i > 0)
    def _():
        o_ref[...] += acc


@functools.lru_cache(maxsize=None)
def _make_segsum(rows, blk):
    grid = rows // blk
    return pl.pallas_call(
        _segsum_body,
        grid=(grid,),
        in_specs=[
            pl.BlockSpec((blk, DH), lambda i: (i, 0)),
            pl.BlockSpec((1, 1, blk), lambda i: (i, 0, 0)),
        ],
        out_specs=pl.BlockSpec((NG, DH), lambda i: (0, 0)),
        out_shape=jax.ShapeDtypeStruct((NG, DH), _F32),
    )


def _heads_body(sum_ref, samp_ref, wp1, bp1, wp2, bp2, wv1, bv1, wv2, bv2,
                v_ref, lp_ref, lg_ref):
    cat = jnp.concatenate([sum_ref[...], samp_ref[...]], axis=-1)
    hv = jax.nn.relu(_dot(cat, wv1[...]) + bv1[...])
    v_ref[...] = _dot(hv, wv2[...]) + bv2[...]
    hp = jax.nn.relu(_dot(cat, wp1[...]) + bp1[...])
    lg = _dot(hp, wp2[...]) + bp2[...]
    lg_ref[...] = lg
    mx = jnp.max(lg, axis=-1, keepdims=True)
    lse = mx + jnp.log(jnp.sum(jnp.exp(lg - mx), axis=-1, keepdims=True))
    lp_ref[...] = lg - lse


_heads_call = pl.pallas_call(
    _heads_body,
    out_shape=[
        jax.ShapeDtypeStruct((NG, 1), _F32),
        jax.ShapeDtypeStruct((NG, NC_OUT), _F32),
        jax.ShapeDtypeStruct((NG, NC_OUT), _F32),
    ],
)


@functools.lru_cache(maxsize=None)
def _make_sc_gather(V, D, blocks, B, nbuf):
    assert blocks % nbuf == 0 or blocks == nbuf or nbuf == 1
    groups = blocks // nbuf
    rows_w = blocks * B
    mesh = plsc.VectorSubcoreMesh(core_axis_name="c", subcore_axis_name="s")

    scratch = [pltpu.VMEM((blocks, B), jnp.int32)]
    scratch += [pltpu.VMEM((B, D), _F32) for _ in range(nbuf)]
    scratch += [pltpu.SemaphoreType.DMA, pltpu.SemaphoreType.DMA]

    @functools.partial(
        pl.kernel, mesh=mesh,
        out_type=jax.ShapeDtypeStruct((NW * rows_w, D), _F32),
        scratch_types=scratch,
    )
    def k(table_hbm, idx_hbm, out_hbm, idx_v, *rest):
        bufs = rest[:nbuf]
        gsem, osem = rest[nbuf], rest[nbuf + 1]
        wid = lax.axis_index("s") * NCORE + lax.axis_index("c")
        base = wid * rows_w
        pltpu.sync_copy(idx_hbm.at[wid], idx_v)

        def group(g, _):
            hs = []
            for b in range(nbuf):
                kb = g * nbuf + b
                hs.append(pltpu.async_copy(
                    table_hbm.at[idx_v.at[kb]], bufs[b], gsem))
            for h in hs:
                h.wait()
            os_ = []
            for b in range(nbuf):
                kb = g * nbuf + b
                os_.append(pltpu.async_copy(
                    bufs[b], out_hbm.at[pl.ds(base + kb * B, B)], osem))
            for o in os_:
                o.wait()
            return _

        lax.fori_loop(0, groups, group, None)

    return k


@functools.lru_cache(maxsize=None)
def _make_sc_scatter_add(EPAD, NROW, blocks, B, nbuf, zchunk):
    ich, iblk = 5, blocks // 5
    assert NSUB * blocks * B == EPAD
    stripe = NROW // NSUB
    assert stripe % zchunk == 0 and stripe % 8 == 0
    zreps = stripe // zchunk
    groups = iblk // nbuf
    mesh = plsc.VectorSubcoreMesh(core_axis_name="c", subcore_axis_name="s",
                                  num_cores=1)

    scratch = [pltpu.VMEM((iblk, B), jnp.int32),
               pltpu.VMEM((zchunk, DH), _F32)]
    scratch += [pltpu.VMEM((B, DH), _F32) for _ in range(nbuf)]
    scratch += [pltpu.VMEM_SHARED((NROW, DH), _F32),
                pltpu.SemaphoreType.DMA]

    @functools.partial(
        pl.kernel, mesh=mesh,
        out_type=jax.ShapeDtypeStruct((NROW, DH), _F32),
        scratch_types=scratch,
    )
    def k(vals_hbm, idx_hbm, out_hbm, idx_v, zbuf, *rest):
        bufs = rest[:nbuf]
        shared, gsem = rest[nbuf], rest[nbuf + 1]
        sid = lax.axis_index("s")

        z16 = jnp.zeros((16,), _F32)

        def zrow(r, _):
            for cc in range(DH // 16):
                zbuf[r, pl.ds(cc * 16, 16)] = z16
            return _

        lax.fori_loop(0, zchunk, zrow, None)
        for rep in range(zreps):
            pltpu.sync_copy(
                zbuf, shared.at[pl.ds(sid * stripe + rep * zchunk, zchunk)])
        plsc.subcore_barrier()

        for c in range(ich):
            pltpu.sync_copy(idx_hbm.at[sid, c], idx_v)
            cbase = sid * blocks * B + c * iblk * B

            def group(g, _):
                hs = []
                for b in range(nbuf):
                    kb = g * nbuf + b
                    hs.append(pltpu.async_copy(
                        vals_hbm.at[pl.ds(cbase + kb * B, B)],
                        bufs[b], gsem))
                for h in hs:
                    h.wait()
                for b in range(nbuf):
                    kb = g * nbuf + b
                    pltpu.sync_copy(bufs[b], shared.at[idx_v.at[kb]], add=True)
                return _

            lax.fori_loop(0, groups, group, None)
        plsc.subcore_barrier()
        pltpu.sync_copy(shared.at[pl.ds(sid * stripe, stripe)],
                        out_hbm.at[pl.ds(sid * stripe, stripe)])

    return k


_EB = 80
_EBLOCKS = E // NW // _EB
_NROW = 10240


def kernel(nodes, edges, senders, receivers, spin_sites, node_graph_idx,
           W_en, b_en, W_ee, b_ee, W_msg, b_msg, W_node, b_node,
           W_edge, b_edge, W_p1, b_p1, W_p2, b_p2, W_v1, b_v1, W_v2, b_v2):
    enc_n = _make_encoder(N, 1000, DF, DH)
    enc_e = _make_encoder(E, 2000, DE, DH)
    proj = _make_proj(N, 1000)
    combine = _make_combine(E, 1000)
    node_upd = _make_node_update(N, 1000)
    gather_p = _make_sc_gather(N, 2 * DH, _EBLOCKS, _EB, 5)
    gather_spin = _make_sc_gather(N, DH, 1, 16, 1)
    scat_edges = _make_sc_scatter_add(E, _NROW, E // NSUB // _EB, _EB, 2, 32)
    segsum = _make_segsum(N, 1000)

    h = enc_n(nodes, W_en, b_en.reshape(1, DH))
    e = enc_e(edges, W_ee, b_ee.reshape(1, DH))

    senders3 = senders.astype(jnp.int32).reshape(NW, _EBLOCKS, _EB)
    receivers3 = receivers.astype(jnp.int32).reshape(NW, _EBLOCKS, _EB)
    recv_scat = receivers.astype(jnp.int32).reshape(
        NSUB, 5, E // NSUB // _EB // 5, _EB)

    Wcat_all = jnp.concatenate(
        [W_msg[:, :DH], W_edge[:, :DH], W_msg[:, DH:2 * DH],
         W_edge[:, DH:2 * DH], W_node[:, :DH]], axis=2)
    Wecat_all = jnp.concatenate([W_msg[:, 2 * DH:], W_edge[:, 2 * DH:]], axis=2)
    bcat_all = jnp.concatenate([b_msg, b_edge], axis=1).reshape(NL, 1, 2 * DH)
    Wa_all = W_node[:, DH:]
    bn_all = b_node.reshape(NL, 1, DH)

    def layer_step(carry, wl):
        h, e = carry
        Wcat, Wecat, bcat, Wa, bn = wl
        Ps, Pr, hW = proj(h, Wcat)
        Gs = gather_p(Ps, senders3)
        Gr = gather_p(Pr, receivers3)
        m, e = combine(Gs, Gr, e, Wecat, bcat)
        agg = scat_edges(m, recv_scat)[:N]
        h = node_upd(hW, h, agg, Wa, bn)
        return (h, e), None

    (h, e), _ = lax.scan(
        layer_step, (h, e), (Wcat_all, Wecat_all, bcat_all, Wa_all, bn_all))

    spin_pad = jnp.concatenate(
        [spin_sites.astype(jnp.int32),
         jnp.zeros((NW * 16 - NG * NSS,), jnp.int32)]).reshape(NW, 1, 16)
    sampled = gather_spin(h, spin_pad)[: NG * NSS].reshape(NG, NSS * DH)

    sum_emb = segsum(h, node_graph_idx.astype(jnp.int32).reshape(10, 1, 1000))

    values, log_prob, logits = _heads_call(
        sum_emb, sampled,
        W_p1, b_p1.reshape(1, -1), W_p2, b_p2.reshape(1, -1),
        W_v1, b_v1.reshape(1, -1), W_v2, b_v2.reshape(1, -1))
    return (values, log_prob, logits)


# speedup vs baseline: 1.7458x; 1.0009x over previous
import functools

import jax
import jax.numpy as jnp
from jax import lax
from jax.experimental import pallas as pl
from jax.experimental.pallas import tpu as pltpu
from jax.experimental.pallas import tpu_sc as plsc

N = 10000
E = 320000
DF = 128
DE = 16
DH = 128
NG = 100
NSS = 5
NC_OUT = 2
NL = 3

NCORE = 2
NSUB = 16
NW = NCORE * NSUB

_F32 = jnp.float32


def _dot(a, b):
    return lax.dot_general(a, b, (((1,), (0,)), ((), ())),
                           precision=lax.Precision.HIGHEST,
                           preferred_element_type=_F32)


def _ln(x):
    mu = jnp.mean(x, axis=-1, keepdims=True)
    var = jnp.mean((x - mu) ** 2, axis=-1, keepdims=True)
    return (x - mu) / jnp.sqrt(var + 1e-6)


def _enc_body(x_ref, w_ref, b_ref, o_ref):
    t = jax.nn.relu(_dot(x_ref[...], w_ref[...]) + b_ref[...])
    o_ref[...] = _ln(t)


@functools.lru_cache(maxsize=None)
def _make_encoder(rows, blk, din, dout):
    grid = rows // blk
    return pl.pallas_call(
        _enc_body,
        grid=(grid,),
        in_specs=[
            pl.BlockSpec((blk, din), lambda i: (i, 0)),
            pl.BlockSpec((din, dout), lambda i: (0, 0)),
            pl.BlockSpec((1, dout), lambda i: (0, 0)),
        ],
        out_specs=pl.BlockSpec((blk, dout), lambda i: (i, 0)),
        out_shape=jax.ShapeDtypeStruct((rows, dout), _F32),
    )


def _proj_body(h_ref, w_ref, ps_ref, pr_ref, hw_ref):
    t = _dot(h_ref[...], w_ref[...])
    ps_ref[...] = t[:, : 2 * DH]
    pr_ref[...] = t[:, 2 * DH: 4 * DH]
    hw_ref[...] = t[:, 4 * DH:]


@functools.lru_cache(maxsize=None)
def _make_proj(rows, blk):
    grid = rows // blk
    return pl.pallas_call(
        _proj_body,
        grid=(grid,),
        in_specs=[
            pl.BlockSpec((blk, DH), lambda i: (i, 0)),
            pl.BlockSpec((DH, 5 * DH), lambda i: (0, 0)),
        ],
        out_specs=[
            pl.BlockSpec((blk, 2 * DH), lambda i: (i, 0)),
            pl.BlockSpec((blk, 2 * DH), lambda i: (i, 0)),
            pl.BlockSpec((blk, DH), lambda i: (i, 0)),
        ],
        out_shape=[
            jax.ShapeDtypeStruct((rows, 2 * DH), _F32),
            jax.ShapeDtypeStruct((rows, 2 * DH), _F32),
            jax.ShapeDtypeStruct((rows, DH), _F32),
        ],
    )


def _combine_body(gs_ref, gr_ref, e_ref, w_ref, b_ref, m_ref, en_ref):
    e = e_ref[...]
    s = gs_ref[...] + gr_ref[...] + _dot(e, w_ref[...]) + b_ref[...]
    m_ref[...] = jax.nn.relu(s[:, :DH])
    en_ref[...] = _ln(jax.nn.relu(s[:, DH:]) + e)


@functools.lru_cache(maxsize=None)
def _make_combine(rows, blk):
    grid = rows // blk
    return pl.pallas_call(
        _combine_body,
        grid=(grid,),
        in_specs=[
            pl.BlockSpec((blk, 2 * DH), lambda i: (i, 0)),
            pl.BlockSpec((blk, 2 * DH), lambda i: (i, 0)),
            pl.BlockSpec((blk, DH), lambda i: (i, 0)),
            pl.BlockSpec((DH, 2 * DH), lambda i: (0, 0)),
            pl.BlockSpec((1, 2 * DH), lambda i: (0, 0)),
        ],
        out_specs=[
            pl.BlockSpec((blk, DH), lambda i: (i, 0)),
            pl.BlockSpec((blk, DH), lambda i: (i, 0)),
        ],
        out_shape=[
            jax.ShapeDtypeStruct((rows, DH), _F32),
            jax.ShapeDtypeStruct((rows, DH), _F32),
        ],
    )


def _node_upd_body(hw_ref, h_ref, agg_ref, wa_ref, b_ref, o_ref):
    h = h_ref[...]
    t = hw_ref[...] + _dot(agg_ref[...], wa_ref[...]) + b_ref[...]
    o_ref[...] = _ln(jax.nn.relu(t) + h)


@functools.lru_cache(maxsize=None)
def _make_node_update(rows, blk):
    grid = rows // blk
    return pl.pallas_call(
        _node_upd_body,
        grid=(grid,),
        in_specs=[
            pl.BlockSpec((blk, DH), lambda i: (i, 0)),
            pl.BlockSpec((blk, DH), lambda i: (i, 0)),
            pl.BlockSpec((blk, DH), lambda i: (i, 0)),
            pl.BlockSpec((DH, DH), lambda i: (0, 0)),
            pl.BlockSpec((1, DH), lambda i: (0, 0)),
        ],
        out_specs=pl.BlockSpec((blk, DH), lambda i: (i, 0)),
        out_shape=jax.ShapeDtypeStruct((rows, DH), _F32),
    )


def _segsum_body(h_ref, nig_ref, o_ref):
    i = pl.program_id(0)
    idx = nig_ref[0, 0, :]
    seg = lax.broadcasted_iota(jnp.int32, (NG, idx.shape[0]), 0)
    onehot = (seg == idx[None, :]).astype(_F32)
    acc = _dot(onehot, h_ref[...])

    @pl.when(i == 0)
    def _():
        o_ref[...] = acc

    @pl.when(i > 0)
    def _():
        o_ref[...] += acc


@functools.lru_cache(maxsize=None)
def _make_segsum(rows, blk):
    grid = rows // blk
    return pl.pallas_call(
        _segsum_body,
        grid=(grid,),
        in_specs=[
            pl.BlockSpec((blk, DH), lambda i: (i, 0)),
            pl.BlockSpec((1, 1, blk), lambda i: (i, 0, 0)),
        ],
        out_specs=pl.BlockSpec((NG, DH), lambda i: (0, 0)),
        out_shape=jax.ShapeDtypeStruct((NG, DH), _F32),
    )


def _heads_body(sum_ref, samp_ref, wp1, bp1, wp2, bp2, wv1, bv1, wv2, bv2,
                v_ref, lp_ref, lg_ref):
    cat = jnp.concatenate([sum_ref[...], samp_ref[...]], axis=-1)
    hv = jax.nn.relu(_dot(cat, wv1[...]) + bv1[...])
    v_ref[...] = _dot(hv, wv2[...]) + bv2[...]
    hp = jax.nn.relu(_dot(cat, wp1[...]) + bp1[...])
    lg = _dot(hp, wp2[...]) + bp2[...]
    lg_ref[...] = lg
    mx = jnp.max(lg, axis=-1, keepdims=True)
    lse = mx + jnp.log(jnp.sum(jnp.exp(lg - mx), axis=-1, keepdims=True))
    lp_ref[...] = lg - lse


_heads_call = pl.pallas_call(
    _heads_body,
    out_shape=[
        jax.ShapeDtypeStruct((NG, 1), _F32),
        jax.ShapeDtypeStruct((NG, NC_OUT), _F32),
        jax.ShapeDtypeStruct((NG, NC_OUT), _F32),
    ],
)


@functools.lru_cache(maxsize=None)
def _make_sc_gather(V, D, blocks, B, nbuf):
    assert blocks % nbuf == 0 or blocks == nbuf or nbuf == 1
    groups = blocks // nbuf
    rows_w = blocks * B
    mesh = plsc.VectorSubcoreMesh(core_axis_name="c", subcore_axis_name="s")

    scratch = [pltpu.VMEM((blocks, B), jnp.int32)]
    scratch += [pltpu.VMEM((B, D), _F32) for _ in range(nbuf)]
    scratch += [pltpu.SemaphoreType.DMA, pltpu.SemaphoreType.DMA]

    @functools.partial(
        pl.kernel, mesh=mesh,
        out_type=jax.ShapeDtypeStruct((NW * rows_w, D), _F32),
        scratch_types=scratch,
    )
    def k(table_hbm, idx_hbm, out_hbm, idx_v, *rest):
        bufs = rest[:nbuf]
        gsem, osem = rest[nbuf], rest[nbuf + 1]
        wid = lax.axis_index("s") * NCORE + lax.axis_index("c")
        base = wid * rows_w
        pltpu.sync_copy(idx_hbm.at[wid], idx_v)

        def group(g, _):
            hs = []
            for b in range(nbuf):
                kb = g * nbuf + b
                hs.append(pltpu.async_copy(
                    table_hbm.at[idx_v.at[kb]], bufs[b], gsem))
            for h in hs:
                h.wait()
            os_ = []
            for b in range(nbuf):
                kb = g * nbuf + b
                os_.append(pltpu.async_copy(
                    bufs[b], out_hbm.at[pl.ds(base + kb * B, B)], osem))
            for o in os_:
                o.wait()
            return _

        lax.fori_loop(0, groups, group, None)

    return k


@functools.lru_cache(maxsize=None)
def _make_sc_gather2(V, D, blocks, B, nbuf):
    ich, iblk = 5, blocks // 5
    groups = iblk // nbuf
    rows_w = blocks * B
    mesh = plsc.VectorSubcoreMesh(core_axis_name="c", subcore_axis_name="s")

    scratch = [pltpu.VMEM((iblk, B), jnp.int32)]
    scratch += [pltpu.VMEM((B, D), _F32) for _ in range(nbuf)]
    scratch += [pltpu.SemaphoreType.DMA, pltpu.SemaphoreType.DMA]

    @functools.partial(
        pl.kernel, mesh=mesh,
        out_type=[jax.ShapeDtypeStruct((NW * rows_w, D), _F32),
                  jax.ShapeDtypeStruct((NW * rows_w, D), _F32)],
        scratch_types=scratch,
    )
    def k(ts_hbm, tr_hbm, idxs_hbm, idxr_hbm, outs_hbm, outr_hbm,
          idx_v, *rest):
        bufs = rest[:nbuf]
        gsem, osem = rest[nbuf], rest[nbuf + 1]
        wid = lax.axis_index("s") * NCORE + lax.axis_index("c")
        base = wid * rows_w

        for idx_hbm, table_hbm, out_hbm in ((idxs_hbm, ts_hbm, outs_hbm),
                                            (idxr_hbm, tr_hbm, outr_hbm)):
            for c in range(ich):
                pltpu.sync_copy(idx_hbm.at[wid, c], idx_v)
                cbase = base + c * iblk * B

                def group(g, _, table_hbm=table_hbm, out_hbm=out_hbm,
                          cbase=cbase):
                    hs = []
                    for b in range(nbuf):
                        kb = g * nbuf + b
                        hs.append(pltpu.async_copy(
                            table_hbm.at[idx_v.at[kb]], bufs[b], gsem))
                    for h in hs:
                        h.wait()
                    os_ = []
                    for b in range(nbuf):
                        kb = g * nbuf + b
                        os_.append(pltpu.async_copy(
                            bufs[b], out_hbm.at[pl.ds(cbase + kb * B, B)],
                            osem))
                    for o in os_:
                        o.wait()
                    return _

                lax.fori_loop(0, groups, group, None)

    return k


def _spin_gather_body(idx_ref, h_ref, o_ref):
    def row(i, _):
        o_ref[pl.ds(i, 1), :] = h_ref[pl.ds(idx_ref[i], 1), :]
        return _

    lax.fori_loop(0, o_ref.shape[0], row, None)


@functools.lru_cache(maxsize=None)
def _make_spin_gather(rows_out):
    return pl.pallas_call(
        _spin_gather_body,
        in_specs=[
            pl.BlockSpec(memory_space=pltpu.SMEM),
            pl.BlockSpec((N, DH), lambda: (0, 0)),
        ],
        out_specs=pl.BlockSpec((rows_out, DH), lambda: (0, 0)),
        out_shape=jax.ShapeDtypeStruct((rows_out, DH), _F32),
    )


@functools.lru_cache(maxsize=None)
def _make_sc_scatter_add(EPAD, NROW, blocks, B, nbuf, zchunk):
    ich, iblk = 5, blocks // 5
    assert NSUB * blocks * B == EPAD
    stripe = NROW // NSUB
    assert stripe % zchunk == 0 and stripe % 8 == 0
    zreps = stripe // zchunk
    groups = iblk // nbuf
    mesh = plsc.VectorSubcoreMesh(core_axis_name="c", subcore_axis_name="s",
                                  num_cores=1)

    scratch = [pltpu.VMEM((iblk, B), jnp.int32),
               pltpu.VMEM((zchunk, DH), _F32)]
    scratch += [pltpu.VMEM((B, DH), _F32) for _ in range(nbuf)]
    scratch += [pltpu.VMEM_SHARED((NROW, DH), _F32),
                pltpu.SemaphoreType.DMA]

    @functools.partial(
        pl.kernel, mesh=mesh,
        out_type=jax.ShapeDtypeStruct((NROW, DH), _F32),
        scratch_types=scratch,
    )
    def k(vals_hbm, idx_hbm, out_hbm, idx_v, zbuf, *rest):
        bufs = rest[:nbuf]
        shared, gsem = rest[nbuf], rest[nbuf + 1]
        sid = lax.axis_index("s")

        z16 = jnp.zeros((16,), _F32)

        def zrow(r, _):
            for cc in range(DH // 16):
                zbuf[r, pl.ds(cc * 16, 16)] = z16
            return _

        lax.fori_loop(0, zchunk, zrow, None)
        for rep in range(zreps):
            pltpu.sync_copy(
                zbuf, shared.at[pl.ds(sid * stripe + rep * zchunk, zchunk)])
        plsc.subcore_barrier()

        for c in range(ich):
            pltpu.sync_copy(idx_hbm.at[sid, c], idx_v)
            cbase = sid * blocks * B + c * iblk * B

            def group(g, _):
                hs = []
                for b in range(nbuf):
                    kb = g * nbuf + b
                    hs.append(pltpu.async_copy(
                        vals_hbm.at[pl.ds(cbase + kb * B, B)],
                        bufs[b], gsem))
                for h in hs:
                    h.wait()
                for b in range(nbuf):
                    kb = g * nbuf + b
                    pltpu.sync_copy(bufs[b], shared.at[idx_v.at[kb]], add=True)
                return _

            lax.fori_loop(0, groups, group, None)
        plsc.subcore_barrier()
        pltpu.sync_copy(shared.at[pl.ds(sid * stripe, stripe)],
                        out_hbm.at[pl.ds(sid * stripe, stripe)])

    return k


_EB = 80
_EBLOCKS = E // NW // _EB
_NROW = 10240


def kernel(nodes, edges, senders, receivers, spin_sites, node_graph_idx,
           W_en, b_en, W_ee, b_ee, W_msg, b_msg, W_node, b_node,
           W_edge, b_edge, W_p1, b_p1, W_p2, b_p2, W_v1, b_v1, W_v2, b_v2):
    enc_n = _make_encoder(N, 1000, DF, DH)
    enc_e = _make_encoder(E, 2000, DE, DH)
    proj = _make_proj(N, 1000)
    combine = _make_combine(E, 1000)
    node_upd = _make_node_update(N, 1000)
    gather2 = _make_sc_gather2(N, 2 * DH, _EBLOCKS, _EB, 5)
    gather_spin = _make_spin_gather(NG * NSS)
    scat_edges = _make_sc_scatter_add(E, _NROW, E // NSUB // _EB, _EB, 2, 32)
    segsum = _make_segsum(N, 1000)

    h = enc_n(nodes, W_en, b_en.reshape(1, DH))
    e = enc_e(edges, W_ee, b_ee.reshape(1, DH))

    senders3 = senders.astype(jnp.int32).reshape(NW, 5, _EBLOCKS // 5, _EB)
    receivers3 = receivers.astype(jnp.int32).reshape(NW, 5, _EBLOCKS // 5, _EB)
    recv_scat = receivers.astype(jnp.int32).reshape(
        NSUB, 5, E // NSUB // _EB // 5, _EB)

    Wcat_all = jnp.concatenate(
        [W_msg[:, :DH], W_edge[:, :DH], W_msg[:, DH:2 * DH],
         W_edge[:, DH:2 * DH], W_node[:, :DH]], axis=2)
    Wecat_all = jnp.concatenate([W_msg[:, 2 * DH:], W_edge[:, 2 * DH:]], axis=2)
    bcat_all = jnp.concatenate([b_msg, b_edge], axis=1).reshape(NL, 1, 2 * DH)
    Wa_all = W_node[:, DH:]
    bn_all = b_node.reshape(NL, 1, DH)

    def layer_step(carry, wl):
        h, e = carry
        Wcat, Wecat, bcat, Wa, bn = wl
        Ps, Pr, hW = proj(h, Wcat)
        Gs, Gr = gather2(Ps, Pr, senders3, receivers3)
        m, e = combine(Gs, Gr, e, Wecat, bcat)
        agg = scat_edges(m, recv_scat)[:N]
        h = node_upd(hW, h, agg, Wa, bn)
        return (h, e), None

    (h, e), _ = lax.scan(
        layer_step, (h, e), (Wcat_all, Wecat_all, bcat_all, Wa_all, bn_all))

    sampled = gather_spin(spin_sites.astype(jnp.int32),
                          h).reshape(NG, NSS * DH)

    sum_emb = segsum(h, node_graph_idx.astype(jnp.int32).reshape(10, 1, 1000))

    values, log_prob, logits = _heads_call(
        sum_emb, sampled,
        W_p1, b_p1.reshape(1, -1), W_p2, b_p2.reshape(1, -1),
        W_v1, b_v1.reshape(1, -1), W_v2, b_v2.reshape(1, -1))
    return (values, log_prob, logits)


# speedup vs baseline: 1.8416x; 1.0549x over previous
import functools

import jax
import jax.numpy as jnp
from jax import lax
from jax.experimental import pallas as pl
from jax.experimental.pallas import tpu as pltpu
from jax.experimental.pallas import tpu_sc as plsc

N = 10000
E = 320000
DF = 128
DE = 16
DH = 128
NG = 100
NSS = 5
NC_OUT = 2
NL = 3

NCORE = 2
NSUB = 16
NW = NCORE * NSUB

_F32 = jnp.float32


def _dot(a, b):
    ah = a.astype(jnp.bfloat16)
    al = (a - ah.astype(_F32)).astype(jnp.bfloat16)
    bh = b.astype(jnp.bfloat16)
    bl = (b - bh.astype(_F32)).astype(jnp.bfloat16)

    def d(x, y):
        return lax.dot_general(x, y, (((1,), (0,)), ((), ())),
                               preferred_element_type=_F32)

    return d(ah, bh) + (d(al, bh) + d(ah, bl))


def _ln(x):
    mu = jnp.mean(x, axis=-1, keepdims=True)
    var = jnp.mean((x - mu) ** 2, axis=-1, keepdims=True)
    return (x - mu) / jnp.sqrt(var + 1e-6)


def _enc_body(x_ref, w_ref, b_ref, o_ref):
    t = jax.nn.relu(_dot(x_ref[...], w_ref[...]) + b_ref[...])
    o_ref[...] = _ln(t)


@functools.lru_cache(maxsize=None)
def _make_encoder(rows, blk, din, dout):
    grid = rows // blk
    return pl.pallas_call(
        _enc_body,
        grid=(grid,),
        in_specs=[
            pl.BlockSpec((blk, din), lambda i: (i, 0)),
            pl.BlockSpec((din, dout), lambda i: (0, 0)),
            pl.BlockSpec((1, dout), lambda i: (0, 0)),
        ],
        out_specs=pl.BlockSpec((blk, dout), lambda i: (i, 0)),
        out_shape=jax.ShapeDtypeStruct((rows, dout), _F32),
    )


def _proj_body(h_ref, w_ref, ps_ref, pr_ref, hw_ref):
    t = _dot(h_ref[...], w_ref[...])
    ps_ref[...] = t[:, : 2 * DH]
    pr_ref[...] = t[:, 2 * DH: 4 * DH]
    hw_ref[...] = t[:, 4 * DH:]


@functools.lru_cache(maxsize=None)
def _make_proj(rows, blk):
    grid = rows // blk
    return pl.pallas_call(
        _proj_body,
        grid=(grid,),
        in_specs=[
            pl.BlockSpec((blk, DH), lambda i: (i, 0)),
            pl.BlockSpec((DH, 5 * DH), lambda i: (0, 0)),
        ],
        out_specs=[
            pl.BlockSpec((blk, 2 * DH), lambda i: (i, 0)),
            pl.BlockSpec((blk, 2 * DH), lambda i: (i, 0)),
            pl.BlockSpec((blk, DH), lambda i: (i, 0)),
        ],
        out_shape=[
            jax.ShapeDtypeStruct((rows, 2 * DH), _F32),
            jax.ShapeDtypeStruct((rows, 2 * DH), _F32),
            jax.ShapeDtypeStruct((rows, DH), _F32),
        ],
    )


def _combine_body(gs_ref, gr_ref, e_ref, w_ref, b_ref, m_ref, en_ref):
    e = e_ref[...]
    s = gs_ref[...] + gr_ref[...] + _dot(e, w_ref[...]) + b_ref[...]
    m_ref[...] = jax.nn.relu(s[:, :DH])
    en_ref[...] = _ln(jax.nn.relu(s[:, DH:]) + e)


@functools.lru_cache(maxsize=None)
def _make_combine(rows, blk):
    grid = rows // blk
    return pl.pallas_call(
        _combine_body,
        grid=(grid,),
        in_specs=[
            pl.BlockSpec((blk, 2 * DH), lambda i: (i, 0)),
            pl.BlockSpec((blk, 2 * DH), lambda i: (i, 0)),
            pl.BlockSpec((blk, DH), lambda i: (i, 0)),
            pl.BlockSpec((DH, 2 * DH), lambda i: (0, 0)),
            pl.BlockSpec((1, 2 * DH), lambda i: (0, 0)),
        ],
        out_specs=[
            pl.BlockSpec((blk, DH), lambda i: (i, 0)),
            pl.BlockSpec((blk, DH), lambda i: (i, 0)),
        ],
        out_shape=[
            jax.ShapeDtypeStruct((rows, DH), _F32),
            jax.ShapeDtypeStruct((rows, DH), _F32),
        ],
    )


def _node_upd_body(hw_ref, h_ref, agg_ref, wa_ref, b_ref, o_ref):
    h = h_ref[...]
    t = hw_ref[...] + _dot(agg_ref[...], wa_ref[...]) + b_ref[...]
    o_ref[...] = _ln(jax.nn.relu(t) + h)


@functools.lru_cache(maxsize=None)
def _make_node_update(rows, blk):
    grid = rows // blk
    return pl.pallas_call(
        _node_upd_body,
        grid=(grid,),
        in_specs=[
            pl.BlockSpec((blk, DH), lambda i: (i, 0)),
            pl.BlockSpec((blk, DH), lambda i: (i, 0)),
            pl.BlockSpec((blk, DH), lambda i: (i, 0)),
            pl.BlockSpec((DH, DH), lambda i: (0, 0)),
            pl.BlockSpec((1, DH), lambda i: (0, 0)),
        ],
        out_specs=pl.BlockSpec((blk, DH), lambda i: (i, 0)),
        out_shape=jax.ShapeDtypeStruct((rows, DH), _F32),
    )


def _segsum_body(h_ref, nig_ref, o_ref):
    i = pl.program_id(0)
    idx = nig_ref[0, 0, :]
    seg = lax.broadcasted_iota(jnp.int32, (NG, idx.shape[0]), 0)
    onehot = (seg == idx[None, :]).astype(_F32)
    acc = _dot(onehot, h_ref[...])

    @pl.when(i == 0)
    def _():
        o_ref[...] = acc

    @pl.when(i > 0)
    def _():
        o_ref[...] += acc


@functools.lru_cache(maxsize=None)
def _make_segsum(rows, blk):
    grid = rows // blk
    return pl.pallas_call(
        _segsum_body,
        grid=(grid,),
        in_specs=[
            pl.BlockSpec((blk, DH), lambda i: (i, 0)),
            pl.BlockSpec((1, 1, blk), lambda i: (i, 0, 0)),
        ],
        out_specs=pl.BlockSpec((NG, DH), lambda i: (0, 0)),
        out_shape=jax.ShapeDtypeStruct((NG, DH), _F32),
    )


def _heads_body(sum_ref, samp_ref, wp1, bp1, wp2, bp2, wv1, bv1, wv2, bv2,
                v_ref, lp_ref, lg_ref):
    cat = jnp.concatenate([sum_ref[...], samp_ref[...]], axis=-1)
    hv = jax.nn.relu(_dot(cat, wv1[...]) + bv1[...])
    v_ref[...] = _dot(hv, wv2[...]) + bv2[...]
    hp = jax.nn.relu(_dot(cat, wp1[...]) + bp1[...])
    lg = _dot(hp, wp2[...]) + bp2[...]
    lg_ref[...] = lg
    mx = jnp.max(lg, axis=-1, keepdims=True)
    lse = mx + jnp.log(jnp.sum(jnp.exp(lg - mx), axis=-1, keepdims=True))
    lp_ref[...] = lg - lse


_heads_call = pl.pallas_call(
    _heads_body,
    out_shape=[
        jax.ShapeDtypeStruct((NG, 1), _F32),
        jax.ShapeDtypeStruct((NG, NC_OUT), _F32),
        jax.ShapeDtypeStruct((NG, NC_OUT), _F32),
    ],
)


@functools.lru_cache(maxsize=None)
def _make_sc_gather(V, D, blocks, B, nbuf):
    assert blocks % nbuf == 0 or blocks == nbuf or nbuf == 1
    groups = blocks // nbuf
    rows_w = blocks * B
    mesh = plsc.VectorSubcoreMesh(core_axis_name="c", subcore_axis_name="s")

    scratch = [pltpu.VMEM((blocks, B), jnp.int32)]
    scratch += [pltpu.VMEM((B, D), _F32) for _ in range(nbuf)]
    scratch += [pltpu.SemaphoreType.DMA, pltpu.SemaphoreType.DMA]

    @functools.partial(
        pl.kernel, mesh=mesh,
        out_type=jax.ShapeDtypeStruct((NW * rows_w, D), _F32),
        scratch_types=scratch,
    )
    def k(table_hbm, idx_hbm, out_hbm, idx_v, *rest):
        bufs = rest[:nbuf]
        gsem, osem = rest[nbuf], rest[nbuf + 1]
        wid = lax.axis_index("s") * NCORE + lax.axis_index("c")
        base = wid * rows_w
        pltpu.sync_copy(idx_hbm.at[wid], idx_v)

        def group(g, _):
            hs = []
            for b in range(nbuf):
                kb = g * nbuf + b
                hs.append(pltpu.async_copy(
                    table_hbm.at[idx_v.at[kb]], bufs[b], gsem))
            for h in hs:
                h.wait()
            os_ = []
            for b in range(nbuf):
                kb = g * nbuf + b
                os_.append(pltpu.async_copy(
                    bufs[b], out_hbm.at[pl.ds(base + kb * B, B)], osem))
            for o in os_:
                o.wait()
            return _

        lax.fori_loop(0, groups, group, None)

    return k


@functools.lru_cache(maxsize=None)
def _make_sc_gather2(V, D, blocks, B, nbuf):
    ich, iblk = 5, blocks // 5
    groups = iblk // nbuf
    rows_w = blocks * B
    mesh = plsc.VectorSubcoreMesh(core_axis_name="c", subcore_axis_name="s")

    scratch = [pltpu.VMEM((iblk, B), jnp.int32)]
    scratch += [pltpu.VMEM((B, D), _F32) for _ in range(nbuf)]
    scratch += [pltpu.SemaphoreType.DMA, pltpu.SemaphoreType.DMA]

    @functools.partial(
        pl.kernel, mesh=mesh,
        out_type=[jax.ShapeDtypeStruct((NW * rows_w, D), _F32),
                  jax.ShapeDtypeStruct((NW * rows_w, D), _F32)],
        scratch_types=scratch,
    )
    def k(ts_hbm, tr_hbm, idxs_hbm, idxr_hbm, outs_hbm, outr_hbm,
          idx_v, *rest):
        bufs = rest[:nbuf]
        gsem, osem = rest[nbuf], rest[nbuf + 1]
        wid = lax.axis_index("s") * NCORE + lax.axis_index("c")
        base = wid * rows_w

        for idx_hbm, table_hbm, out_hbm in ((idxs_hbm, ts_hbm, outs_hbm),
                                            (idxr_hbm, tr_hbm, outr_hbm)):
            for c in range(ich):
                pltpu.sync_copy(idx_hbm.at[wid, c], idx_v)
                cbase = base + c * iblk * B

                def group(g, _, table_hbm=table_hbm, out_hbm=out_hbm,
                          cbase=cbase):
                    hs = []
                    for b in range(nbuf):
                        kb = g * nbuf + b
                        hs.append(pltpu.async_copy(
                            table_hbm.at[idx_v.at[kb]], bufs[b], gsem))
                    for h in hs:
                        h.wait()
                    os_ = []
                    for b in range(nbuf):
                        kb = g * nbuf + b
                        os_.append(pltpu.async_copy(
                            bufs[b], out_hbm.at[pl.ds(cbase + kb * B, B)],
                            osem))
                    for o in os_:
                        o.wait()
                    return _

                lax.fori_loop(0, groups, group, None)

    return k


def _spin_gather_body(idx_ref, h_ref, o_ref):
    def row(i, _):
        o_ref[pl.ds(i, 1), :] = h_ref[pl.ds(idx_ref[i], 1), :]
        return _

    lax.fori_loop(0, o_ref.shape[0], row, None)


@functools.lru_cache(maxsize=None)
def _make_spin_gather(rows_out):
    return pl.pallas_call(
        _spin_gather_body,
        in_specs=[
            pl.BlockSpec(memory_space=pltpu.SMEM),
            pl.BlockSpec((N, DH), lambda: (0, 0)),
        ],
        out_specs=pl.BlockSpec((rows_out, DH), lambda: (0, 0)),
        out_shape=jax.ShapeDtypeStruct((rows_out, DH), _F32),
    )


@functools.lru_cache(maxsize=None)
def _make_sc_scatter_add(EPAD, NROW, blocks, B, nbuf, zchunk):
    ich, iblk = 5, blocks // 5
    assert NSUB * blocks * B == EPAD
    stripe = NROW // NSUB
    assert stripe % zchunk == 0 and stripe % 8 == 0
    zreps = stripe // zchunk
    groups = iblk // nbuf
    mesh = plsc.VectorSubcoreMesh(core_axis_name="c", subcore_axis_name="s",
                                  num_cores=1)

    scratch = [pltpu.VMEM((iblk, B), jnp.int32),
               pltpu.VMEM((zchunk, DH), _F32)]
    scratch += [pltpu.VMEM((B, DH), _F32) for _ in range(nbuf)]
    scratch += [pltpu.VMEM_SHARED((NROW, DH), _F32),
                pltpu.SemaphoreType.DMA]

    @functools.partial(
        pl.kernel, mesh=mesh,
        out_type=jax.ShapeDtypeStruct((NROW, DH), _F32),
        scratch_types=scratch,
    )
    def k(vals_hbm, idx_hbm, out_hbm, idx_v, zbuf, *rest):
        bufs = rest[:nbuf]
        shared, gsem = rest[nbuf], rest[nbuf + 1]
        sid = lax.axis_index("s")

        z16 = jnp.zeros((16,), _F32)

        def zrow(r, _):
            for cc in range(DH // 16):
                zbuf[r, pl.ds(cc * 16, 16)] = z16
            return _

        lax.fori_loop(0, zchunk, zrow, None)
        for rep in range(zreps):
            pltpu.sync_copy(
                zbuf, shared.at[pl.ds(sid * stripe + rep * zchunk, zchunk)])
        plsc.subcore_barrier()

        for c in range(ich):
            pltpu.sync_copy(idx_hbm.at[sid, c], idx_v)
            cbase = sid * blocks * B + c * iblk * B

            def group(g, _):
                hs = []
                for b in range(nbuf):
                    kb = g * nbuf + b
                    hs.append(pltpu.async_copy(
                        vals_hbm.at[pl.ds(cbase + kb * B, B)],
                        bufs[b], gsem))
                for h in hs:
                    h.wait()
                for b in range(nbuf):
                    kb = g * nbuf + b
                    pltpu.sync_copy(bufs[b], shared.at[idx_v.at[kb]], add=True)
                return _

            lax.fori_loop(0, groups, group, None)
        plsc.subcore_barrier()
        pltpu.sync_copy(shared.at[pl.ds(sid * stripe, stripe)],
                        out_hbm.at[pl.ds(sid * stripe, stripe)])

    return k


_EB = 80
_EBLOCKS = E // NW // _EB
_NROW = 10240


def kernel(nodes, edges, senders, receivers, spin_sites, node_graph_idx,
           W_en, b_en, W_ee, b_ee, W_msg, b_msg, W_node, b_node,
           W_edge, b_edge, W_p1, b_p1, W_p2, b_p2, W_v1, b_v1, W_v2, b_v2):
    enc_n = _make_encoder(N, 1000, DF, DH)
    enc_e = _make_encoder(E, 2000, DE, DH)
    proj = _make_proj(N, 1000)
    combine = _make_combine(E, 1000)
    node_upd = _make_node_update(N, 1000)
    gather2 = _make_sc_gather2(N, 2 * DH, _EBLOCKS, _EB, 5)
    gather_spin = _make_spin_gather(NG * NSS)
    scat_edges = _make_sc_scatter_add(E, _NROW, E // NSUB // _EB, _EB, 2, 32)
    segsum = _make_segsum(N, 1000)

    h = enc_n(nodes, W_en, b_en.reshape(1, DH))
    e = enc_e(edges, W_ee, b_ee.reshape(1, DH))

    senders3 = senders.astype(jnp.int32).reshape(NW, 5, _EBLOCKS // 5, _EB)
    receivers3 = receivers.astype(jnp.int32).reshape(NW, 5, _EBLOCKS // 5, _EB)
    recv_scat = receivers.astype(jnp.int32).reshape(
        NSUB, 5, E // NSUB // _EB // 5, _EB)

    Wcat_all = jnp.concatenate(
        [W_msg[:, :DH], W_edge[:, :DH], W_msg[:, DH:2 * DH],
         W_edge[:, DH:2 * DH], W_node[:, :DH]], axis=2)
    Wecat_all = jnp.concatenate([W_msg[:, 2 * DH:], W_edge[:, 2 * DH:]], axis=2)
    bcat_all = jnp.concatenate([b_msg, b_edge], axis=1).reshape(NL, 1, 2 * DH)
    Wa_all = W_node[:, DH:]
    bn_all = b_node.reshape(NL, 1, DH)

    def layer_step(carry, wl):
        h, e = carry
        Wcat, Wecat, bcat, Wa, bn = wl
        Ps, Pr, hW = proj(h, Wcat)
        Gs, Gr = gather2(Ps, Pr, senders3, receivers3)
        m, e = combine(Gs, Gr, e, Wecat, bcat)
        agg = scat_edges(m, recv_scat)[:N]
        h = node_upd(hW, h, agg, Wa, bn)
        return (h, e), None

    (h, e), _ = lax.scan(
        layer_step, (h, e), (Wcat_all, Wecat_all, bcat_all, Wa_all, bn_all))

    sampled = gather_spin(spin_sites.astype(jnp.int32),
                          h).reshape(NG, NSS * DH)

    sum_emb = segsum(h, node_graph_idx.astype(jnp.int32).reshape(10, 1, 1000))

    values, log_prob, logits = _heads_call(
        sum_emb, sampled,
        W_p1, b_p1.reshape(1, -1), W_p2, b_p2.reshape(1, -1),
        W_v1, b_v1.reshape(1, -1), W_v2, b_v2.reshape(1, -1))
    return (values, log_prob, logits)


# speedup vs baseline: 1.8490x; 1.0040x over previous
import functools

import jax
import jax.numpy as jnp
from jax import lax
from jax.experimental import pallas as pl
from jax.experimental.pallas import tpu as pltpu
from jax.experimental.pallas import tpu_sc as plsc

N = 10000
E = 320000
DF = 128
DE = 16
DH = 128
NG = 100
NSS = 5
NC_OUT = 2
NL = 3

NCORE = 2
NSUB = 16
NW = NCORE * NSUB

_F32 = jnp.float32


def _dot(a, b):
    ah = a.astype(jnp.bfloat16)
    al = (a - ah.astype(_F32)).astype(jnp.bfloat16)
    bh = b.astype(jnp.bfloat16)
    bl = (b - bh.astype(_F32)).astype(jnp.bfloat16)

    def d(x, y):
        return lax.dot_general(x, y, (((1,), (0,)), ((), ())),
                               preferred_element_type=_F32)

    return d(ah, bh) + (d(al, bh) + d(ah, bl))


def _ln(x):
    mu = jnp.mean(x, axis=-1, keepdims=True)
    var = jnp.mean((x - mu) ** 2, axis=-1, keepdims=True)
    return (x - mu) / jnp.sqrt(var + 1e-6)


def _enc_body(x_ref, w_ref, b_ref, o_ref):
    t = jax.nn.relu(_dot(x_ref[...], w_ref[...]) + b_ref[...])
    o_ref[...] = _ln(t)


@functools.lru_cache(maxsize=None)
def _make_encoder(rows, blk, din, dout):
    grid = rows // blk
    return pl.pallas_call(
        _enc_body,
        grid=(grid,),
        in_specs=[
            pl.BlockSpec((blk, din), lambda i: (i, 0)),
            pl.BlockSpec((din, dout), lambda i: (0, 0)),
            pl.BlockSpec((1, dout), lambda i: (0, 0)),
        ],
        out_specs=pl.BlockSpec((blk, dout), lambda i: (i, 0)),
        out_shape=jax.ShapeDtypeStruct((rows, dout), _F32),
    )


def _proj_body(h_ref, w_ref, ps_ref, pr_ref, hw_ref):
    t = _dot(h_ref[...], w_ref[...])
    ps_ref[...] = t[:, : 2 * DH]
    pr_ref[...] = t[:, 2 * DH: 4 * DH]
    hw_ref[...] = t[:, 4 * DH:]


@functools.lru_cache(maxsize=None)
def _make_proj(rows, blk):
    grid = rows // blk
    return pl.pallas_call(
        _proj_body,
        grid=(grid,),
        in_specs=[
            pl.BlockSpec((blk, DH), lambda i: (i, 0)),
            pl.BlockSpec((DH, 5 * DH), lambda i: (0, 0)),
        ],
        out_specs=[
            pl.BlockSpec((blk, 2 * DH), lambda i: (i, 0)),
            pl.BlockSpec((blk, 2 * DH), lambda i: (i, 0)),
            pl.BlockSpec((blk, DH), lambda i: (i, 0)),
        ],
        out_shape=[
            jax.ShapeDtypeStruct((rows, 2 * DH), _F32),
            jax.ShapeDtypeStruct((rows, 2 * DH), _F32),
            jax.ShapeDtypeStruct((rows, DH), _F32),
        ],
    )


def _combine_m_body(gs_ref, gr_ref, e_ref, w_ref, b_ref, m_ref):
    s = gs_ref[...] + gr_ref[...] + _dot(e_ref[...], w_ref[...]) + b_ref[...]
    m_ref[...] = jax.nn.relu(s)


def _combine_e_body(gs_ref, gr_ref, e_ref, w_ref, b_ref, en_ref):
    e = e_ref[...]
    s = gs_ref[...] + gr_ref[...] + _dot(e, w_ref[...]) + b_ref[...]
    en_ref[...] = _ln(jax.nn.relu(s) + e)


@functools.lru_cache(maxsize=None)
def _make_combine(rows, blk, part):
    grid = rows // blk
    body = _combine_m_body if part == 0 else _combine_e_body
    return pl.pallas_call(
        body,
        grid=(grid,),
        in_specs=[
            pl.BlockSpec((blk, DH), lambda i: (i, part)),
            pl.BlockSpec((blk, DH), lambda i: (i, part)),
            pl.BlockSpec((blk, DH), lambda i: (i, 0)),
            pl.BlockSpec((DH, DH), lambda i: (0, part)),
            pl.BlockSpec((1, DH), lambda i: (0, part)),
        ],
        out_specs=pl.BlockSpec((blk, DH), lambda i: (i, 0)),
        out_shape=jax.ShapeDtypeStruct((rows, DH), _F32),
    )


def _node_upd_body(hw_ref, h_ref, agg_ref, wa_ref, b_ref, o_ref):
    h = h_ref[...]
    t = hw_ref[...] + _dot(agg_ref[...], wa_ref[...]) + b_ref[...]
    o_ref[...] = _ln(jax.nn.relu(t) + h)


@functools.lru_cache(maxsize=None)
def _make_node_update(rows, blk):
    grid = rows // blk
    return pl.pallas_call(
        _node_upd_body,
        grid=(grid,),
        in_specs=[
            pl.BlockSpec((blk, DH), lambda i: (i, 0)),
            pl.BlockSpec((blk, DH), lambda i: (i, 0)),
            pl.BlockSpec((blk, DH), lambda i: (i, 0)),
            pl.BlockSpec((DH, DH), lambda i: (0, 0)),
            pl.BlockSpec((1, DH), lambda i: (0, 0)),
        ],
        out_specs=pl.BlockSpec((blk, DH), lambda i: (i, 0)),
        out_shape=jax.ShapeDtypeStruct((rows, DH), _F32),
    )


def _segsum_body(h_ref, nig_ref, o_ref):
    i = pl.program_id(0)
    idx = nig_ref[0, 0, :]
    seg = lax.broadcasted_iota(jnp.int32, (NG, idx.shape[0]), 0)
    onehot = (seg == idx[None, :]).astype(_F32)
    acc = _dot(onehot, h_ref[...])

    @pl.when(i == 0)
    def _():
        o_ref[...] = acc

    @pl.when(i > 0)
    def _():
        o_ref[...] += acc


@functools.lru_cache(maxsize=None)
def _make_segsum(rows, blk):
    grid = rows // blk
    return pl.pallas_call(
        _segsum_body,
        grid=(grid,),
        in_specs=[
            pl.BlockSpec((blk, DH), lambda i: (i, 0)),
            pl.BlockSpec((1, 1, blk), lambda i: (i, 0, 0)),
        ],
        out_specs=pl.BlockSpec((NG, DH), lambda i: (0, 0)),
        out_shape=jax.ShapeDtypeStruct((NG, DH), _F32),
    )


def _heads_body(sum_ref, samp_ref, wp1, bp1, wp2, bp2, wv1, bv1, wv2, bv2,
                v_ref, lp_ref, lg_ref):
    cat = jnp.concatenate([sum_ref[...], samp_ref[...]], axis=-1)
    hv = jax.nn.relu(_dot(cat, wv1[...]) + bv1[...])
    v_ref[...] = _dot(hv, wv2[...]) + bv2[...]
    hp = jax.nn.relu(_dot(cat, wp1[...]) + bp1[...])
    lg = _dot(hp, wp2[...]) + bp2[...]
    lg_ref[...] = lg
    mx = jnp.max(lg, axis=-1, keepdims=True)
    lse = mx + jnp.log(jnp.sum(jnp.exp(lg - mx), axis=-1, keepdims=True))
    lp_ref[...] = lg - lse


_heads_call = pl.pallas_call(
    _heads_body,
    out_shape=[
        jax.ShapeDtypeStruct((NG, 1), _F32),
        jax.ShapeDtypeStruct((NG, NC_OUT), _F32),
        jax.ShapeDtypeStruct((NG, NC_OUT), _F32),
    ],
)


@functools.lru_cache(maxsize=None)
def _make_sc_gather(V, D, blocks, B, nbuf):
    assert blocks % nbuf == 0 or blocks == nbuf or nbuf == 1
    groups = blocks // nbuf
    rows_w = blocks * B
    mesh = plsc.VectorSubcoreMesh(core_axis_name="c", subcore_axis_name="s")

    scratch = [pltpu.VMEM((blocks, B), jnp.int32)]
    scratch += [pltpu.VMEM((B, D), _F32) for _ in range(nbuf)]
    scratch += [pltpu.SemaphoreType.DMA, pltpu.SemaphoreType.DMA]

    @functools.partial(
        pl.kernel, mesh=mesh,
        out_type=jax.ShapeDtypeStruct((NW * rows_w, D), _F32),
        scratch_types=scratch,
    )
    def k(table_hbm, idx_hbm, out_hbm, idx_v, *rest):
        bufs = rest[:nbuf]
        gsem, osem = rest[nbuf], rest[nbuf + 1]
        wid = lax.axis_index("s") * NCORE + lax.axis_index("c")
        base = wid * rows_w
        pltpu.sync_copy(idx_hbm.at[wid], idx_v)

        def group(g, _):
            hs = []
            for b in range(nbuf):
                kb = g * nbuf + b
                hs.append(pltpu.async_copy(
                    table_hbm.at[idx_v.at[kb]], bufs[b], gsem))
            for h in hs:
                h.wait()
            os_ = []
            for b in range(nbuf):
                kb = g * nbuf + b
                os_.append(pltpu.async_copy(
                    bufs[b], out_hbm.at[pl.ds(base + kb * B, B)], osem))
            for o in os_:
                o.wait()
            return _

        lax.fori_loop(0, groups, group, None)

    return k


@functools.lru_cache(maxsize=None)
def _make_sc_gather2(V, D, blocks, B, nbuf):
    ich, iblk = 5, blocks // 5
    groups = iblk // nbuf
    rows_w = blocks * B
    mesh = plsc.VectorSubcoreMesh(core_axis_name="c", subcore_axis_name="s")

    scratch = [pltpu.VMEM((iblk, B), jnp.int32)]
    scratch += [pltpu.VMEM((B, D), _F32) for _ in range(nbuf)]
    scratch += [pltpu.SemaphoreType.DMA, pltpu.SemaphoreType.DMA]

    @functools.partial(
        pl.kernel, mesh=mesh,
        out_type=[jax.ShapeDtypeStruct((NW * rows_w, D), _F32),
                  jax.ShapeDtypeStruct((NW * rows_w, D), _F32)],
        scratch_types=scratch,
    )
    def k(ts_hbm, tr_hbm, idxs_hbm, idxr_hbm, outs_hbm, outr_hbm,
          idx_v, *rest):
        bufs = rest[:nbuf]
        gsem, osem = rest[nbuf], rest[nbuf + 1]
        wid = lax.axis_index("s") * NCORE + lax.axis_index("c")
        base = wid * rows_w

        for idx_hbm, table_hbm, out_hbm in ((idxs_hbm, ts_hbm, outs_hbm),
                                            (idxr_hbm, tr_hbm, outr_hbm)):
            for c in range(ich):
                pltpu.sync_copy(idx_hbm.at[wid, c], idx_v)
                cbase = base + c * iblk * B

                def group(g, _, table_hbm=table_hbm, out_hbm=out_hbm,
                          cbase=cbase):
                    hs = []
                    for b in range(nbuf):
                        kb = g * nbuf + b
                        hs.append(pltpu.async_copy(
                            table_hbm.at[idx_v.at[kb]], bufs[b], gsem))
                    for h in hs:
                        h.wait()
                    os_ = []
                    for b in range(nbuf):
                        kb = g * nbuf + b
                        os_.append(pltpu.async_copy(
                            bufs[b], out_hbm.at[pl.ds(cbase + kb * B, B)],
                            osem))
                    for o in os_:
                        o.wait()
                    return _

                lax.fori_loop(0, groups, group, None)

    return k


def _spin_gather_body(idx_ref, h_ref, o_ref):
    def row(i, _):
        o_ref[pl.ds(i, 1), :] = h_ref[pl.ds(idx_ref[i], 1), :]
        return _

    lax.fori_loop(0, o_ref.shape[0], row, None)


@functools.lru_cache(maxsize=None)
def _make_spin_gather(rows_out):
    return pl.pallas_call(
        _spin_gather_body,
        in_specs=[
            pl.BlockSpec(memory_space=pltpu.SMEM),
            pl.BlockSpec((N, DH), lambda: (0, 0)),
        ],
        out_specs=pl.BlockSpec((rows_out, DH), lambda: (0, 0)),
        out_shape=jax.ShapeDtypeStruct((rows_out, DH), _F32),
    )


@functools.lru_cache(maxsize=None)
def _make_sc_scatter_add(EPAD, NROW, blocks, B, nbuf, zchunk):
    ich, iblk = 5, blocks // 5
    assert NSUB * blocks * B == EPAD
    stripe = NROW // NSUB
    assert stripe % zchunk == 0 and stripe % 8 == 0
    zreps = stripe // zchunk
    groups = iblk // nbuf
    mesh = plsc.VectorSubcoreMesh(core_axis_name="c", subcore_axis_name="s",
                                  num_cores=1)

    scratch = [pltpu.VMEM((iblk, B), jnp.int32),
               pltpu.VMEM((zchunk, DH), _F32)]
    scratch += [pltpu.VMEM((B, DH), _F32) for _ in range(nbuf)]
    scratch += [pltpu.VMEM_SHARED((NROW, DH), _F32),
                pltpu.SemaphoreType.DMA]

    @functools.partial(
        pl.kernel, mesh=mesh,
        out_type=jax.ShapeDtypeStruct((NROW, DH), _F32),
        scratch_types=scratch,
    )
    def k(vals_hbm, idx_hbm, out_hbm, idx_v, zbuf, *rest):
        bufs = rest[:nbuf]
        shared, gsem = rest[nbuf], rest[nbuf + 1]
        sid = lax.axis_index("s")

        z16 = jnp.zeros((16,), _F32)

        def zrow(r, _):
            for cc in range(DH // 16):
                zbuf[r, pl.ds(cc * 16, 16)] = z16
            return _

        lax.fori_loop(0, zchunk, zrow, None)
        for rep in range(zreps):
            pltpu.sync_copy(
                zbuf, shared.at[pl.ds(sid * stripe + rep * zchunk, zchunk)])
        plsc.subcore_barrier()

        for c in range(ich):
            pltpu.sync_copy(idx_hbm.at[sid, c], idx_v)
            cbase = sid * blocks * B + c * iblk * B

            def group(g, _):
                hs = []
                for b in range(nbuf):
                    kb = g * nbuf + b
                    hs.append(pltpu.async_copy(
                        vals_hbm.at[pl.ds(cbase + kb * B, B)],
                        bufs[b], gsem))
                for h in hs:
                    h.wait()
                for b in range(nbuf):
                    kb = g * nbuf + b
                    pltpu.sync_copy(bufs[b], shared.at[idx_v.at[kb]], add=True)
                return _

            lax.fori_loop(0, groups, group, None)
        plsc.subcore_barrier()
        pltpu.sync_copy(shared.at[pl.ds(sid * stripe, stripe)],
                        out_hbm.at[pl.ds(sid * stripe, stripe)])

    return k


_EB = 80
_EBLOCKS = E // NW // _EB
_NROW = 10240


def kernel(nodes, edges, senders, receivers, spin_sites, node_graph_idx,
           W_en, b_en, W_ee, b_ee, W_msg, b_msg, W_node, b_node,
           W_edge, b_edge, W_p1, b_p1, W_p2, b_p2, W_v1, b_v1, W_v2, b_v2):
    enc_n = _make_encoder(N, 1000, DF, DH)
    enc_e = _make_encoder(E, 2000, DE, DH)
    proj = _make_proj(N, 1000)
    combine_m = _make_combine(E, 1000, 0)
    combine_e = _make_combine(E, 1000, 1)
    node_upd = _make_node_update(N, 1000)
    gather2 = _make_sc_gather2(N, 2 * DH, _EBLOCKS, _EB, 5)
    gather_spin = _make_spin_gather(NG * NSS)
    scat_edges = _make_sc_scatter_add(E, _NROW, E // NSUB // _EB, _EB, 2, 32)
    segsum = _make_segsum(N, 1000)

    h = enc_n(nodes, W_en, b_en.reshape(1, DH))
    e = enc_e(edges, W_ee, b_ee.reshape(1, DH))

    senders3 = senders.astype(jnp.int32).reshape(NW, 5, _EBLOCKS // 5, _EB)
    receivers3 = receivers.astype(jnp.int32).reshape(NW, 5, _EBLOCKS // 5, _EB)
    recv_scat = receivers.astype(jnp.int32).reshape(
        NSUB, 5, E // NSUB // _EB // 5, _EB)

    Wcat_all = jnp.concatenate(
        [W_msg[:, :DH], W_edge[:, :DH], W_msg[:, DH:2 * DH],
         W_edge[:, DH:2 * DH], W_node[:, :DH]], axis=2)
    Wecat_all = jnp.concatenate([W_msg[:, 2 * DH:], W_edge[:, 2 * DH:]], axis=2)
    bcat_all = jnp.concatenate([b_msg, b_edge], axis=1).reshape(NL, 1, 2 * DH)
    Wa_all = W_node[:, DH:]
    bn_all = b_node.reshape(NL, 1, DH)

    def layer_step(carry, wl):
        h, e = carry
        Wcat, Wecat, bcat, Wa, bn = wl
        Ps, Pr, hW = proj(h, Wcat)
        Gs, Gr = gather2(Ps, Pr, senders3, receivers3)
        m = combine_m(Gs, Gr, e, Wecat, bcat)
        agg = scat_edges(m, recv_scat)[:N]
        e = combine_e(Gs, Gr, e, Wecat, bcat)
        h = node_upd(hW, h, agg, Wa, bn)
        return (h, e), None

    (h, e), _ = lax.scan(
        layer_step, (h, e), (Wcat_all, Wecat_all, bcat_all, Wa_all, bn_all))

    sampled = gather_spin(spin_sites.astype(jnp.int32),
                          h).reshape(NG, NSS * DH)

    sum_emb = segsum(h, node_graph_idx.astype(jnp.int32).reshape(10, 1, 1000))

    values, log_prob, logits = _heads_call(
        sum_emb, sampled,
        W_p1, b_p1.reshape(1, -1), W_p2, b_p2.reshape(1, -1),
        W_v1, b_v1.reshape(1, -1), W_v2, b_v2.reshape(1, -1))
    return (values, log_prob, logits)
